# baseline (speedup 1.0000x reference)
import functools

import jax
import jax.numpy as jnp
from jax import lax
from jax.experimental import pallas as pl
from jax.experimental.pallas import tpu as pltpu
from jax.experimental.pallas import tpu_sc as plsc

VAR_LEN = 100000
EMBED = 64
B = 4096
L = 200

_info = plsc.get_sparse_core_info()
NC, NS, NL = _info.num_cores, _info.num_subcores, _info.num_lanes
NW = NC * NS
BT = B // 128
ET = EMBED // 8
LT = L // 8
NBUF = 2


def _sc_body(t_hbm, idx_hbm, tab_hbm, out_hbm, trow, idxb, sbuf,
             sem_i, sem_s, sem_o, sem_t):
    cid = lax.axis_index("c")
    sid = lax.axis_index("s")
    wid = sid * NC + cid
    e_t = lax.div(wid, 4)
    r0 = 2 * lax.rem(wid, 4)

    def idx_load(l, k):
        return pltpu.make_async_copy(
            idx_hbm.at[lax.div(l, 8), :, lax.rem(l, 8), :], idxb[k], sem_i[k]
        )

    def src_load(l, k):
        return pltpu.make_async_copy(
            t_hbm.at[l * ET + e_t, :, pl.ds(r0, 2), :], sbuf[k], sem_s[k]
        )

    def out_store(l, k):
        return pltpu.make_async_copy(
            sbuf[k], out_hbm.at[l * ET + e_t, :, pl.ds(r0, 2), :], sem_o[k]
        )

    trow_copy = pltpu.make_async_copy(tab_hbm.at[wid], trow, sem_t)
    trow_copy.start()
    idx_load(0, 0).start()
    src_load(0, 0).start()
    trow_copy.wait()

    def outer(h, carry):
        for k in range(NBUF):
            l = NBUF * h + k
            kn = k ^ 1
            idx_load(l, k).wait()
            src_load(l, k).wait()

            @pl.when(l > 0)
            def _():
                out_store(l - 1, kn).wait()

            @pl.when(l + 1 < L)
            def _():
                idx_load(l + 1, kn).start()
                src_load(l + 1, kn).start()

            @plsc.parallel_loop(0, BT, unroll=2)
            def _(r):
                for u in range(128 // NL):
                    ds = pl.ds(u * NL, NL)
                    iv = idxb[k][r, ds]
                    gi = plsc.load_gather(trow, [iv])
                    g0 = plsc.bitcast(jnp.left_shift(gi, 16), jnp.float32)
                    g1 = plsc.bitcast(
                        jnp.bitwise_and(gi, jnp.int32(-65536)), jnp.float32
                    )
                    sbuf[k][r, 0, ds] = sbuf[k][r, 0, ds] + g0
                    sbuf[k][r, 1, ds] = sbuf[k][r, 1, ds] + g1

            out_store(l, k).start()
        return carry

    lax.fori_loop(0, L // NBUF, outer, 0)
    out_store(L - 1, 1).wait()


@jax.jit
def _run(t4, idx4, tabp):
    mesh = plsc.VectorSubcoreMesh(core_axis_name="c", subcore_axis_name="s")
    scratch = [
        pltpu.VMEM((VAR_LEN,), jnp.int32),
        [pltpu.VMEM((BT, 128), jnp.int32) for _ in range(NBUF)],
        [pltpu.VMEM((BT, 2, 128), jnp.float32) for _ in range(NBUF)],
        [pltpu.SemaphoreType.DMA for _ in range(NBUF)],
        [pltpu.SemaphoreType.DMA for _ in range(NBUF)],
        [pltpu.SemaphoreType.DMA for _ in range(NBUF)],
        pltpu.SemaphoreType.DMA,
    ]
    f = functools.partial(
        pl.kernel,
        out_type=jax.ShapeDtypeStruct((L * ET, BT, 8, 128), jnp.float32),
        mesh=mesh,
        scratch_types=scratch,
        compiler_params=pltpu.CompilerParams(
            use_tc_tiling_on_sc=False, needs_layout_passes=False
        ),
    )(_sc_body)
    return f(t4, idx4, tabp)


def kernel(src, variable_seq, emb_weight):
    t4 = (
        src.transpose(1, 2, 0)
        .reshape(L, ET, 8, BT, 128)
        .transpose(0, 1, 3, 2, 4)
        .reshape(L * ET, BT, 8, 128)
    )
    idx4 = (
        variable_seq.astype(jnp.int32)
        .transpose(1, 0)
        .reshape(LT, 8, BT, 128)
        .transpose(0, 2, 1, 3)
    )
    tabp = jax.lax.bitcast_convert_type(
        emb_weight.astype(jnp.bfloat16).reshape(VAR_LEN, NW, 2), jnp.int32
    ).transpose(1, 0)
    out4 = _run(t4, idx4, tabp)
    return (
        out4.reshape(L, ET, BT, 8, 128)
        .transpose(0, 1, 3, 2, 4)
        .reshape(L, EMBED, B)
        .transpose(2, 0, 1)
    )


# speedup vs baseline: 1.0021x; 1.0021x over previous
import functools

import jax
import jax.numpy as jnp
from jax import lax
from jax.experimental import pallas as pl
from jax.experimental.pallas import tpu as pltpu
from jax.experimental.pallas import tpu_sc as plsc

VAR_LEN = 100000
EMBED = 64
B = 4096
L = 200

_info = plsc.get_sparse_core_info()
NC, NS, NL = _info.num_cores, _info.num_subcores, _info.num_lanes
NW = NC * NS
BT = B // 128
ET = EMBED // 8
LT = L // 8
NBUF = 2


def _sc_body(t_hbm, idx_hbm, tab_hbm, out_hbm, trow, idxb, sbuf,
             sem_i, sem_s, sem_o, sem_t):
    cid = lax.axis_index("c")
    sid = lax.axis_index("s")
    wid = sid * NC + cid
    e_t = lax.div(wid, 4)
    r0 = 2 * lax.rem(wid, 4)

    def idx_load(l, k):
        return pltpu.make_async_copy(
            idx_hbm.at[lax.div(l, 8), :, lax.rem(l, 8), :], idxb[k], sem_i[k]
        )

    def src_load(l, k):
        return pltpu.make_async_copy(
            t_hbm.at[l * ET + e_t, :, pl.ds(r0 * 128, 256)], sbuf[k],
            sem_s[k],
        )

    def out_store(l, k):
        return pltpu.make_async_copy(
            sbuf[k], out_hbm.at[l * ET + e_t, :, pl.ds(r0 * 128, 256)],
            sem_o[k],
        )

    trow_copy = pltpu.make_async_copy(tab_hbm.at[wid], trow, sem_t)
    trow_copy.start()
    idx_load(0, 0).start()
    src_load(0, 0).start()
    trow_copy.wait()

    def outer(h, carry):
        for k in range(NBUF):
            l = NBUF * h + k
            kn = k ^ 1
            idx_load(l, k).wait()
            src_load(l, k).wait()

            @pl.when(l > 0)
            def _():
                out_store(l - 1, kn).wait()

            @pl.when(l + 1 < L)
            def _():
                idx_load(l + 1, kn).start()
                src_load(l + 1, kn).start()

            @plsc.parallel_loop(0, BT, unroll=2)
            def _(r):
                for u in range(128 // NL):
                    ds = pl.ds(u * NL, NL)
                    iv = idxb[k][r, ds]
                    gi = plsc.load_gather(trow, [iv])
                    g0 = plsc.bitcast(jnp.left_shift(gi, 16), jnp.float32)
                    g1 = plsc.bitcast(
                        jnp.bitwise_and(gi, jnp.int32(-65536)), jnp.float32
                    )
                    ds1 = pl.ds(128 + u * NL, NL)
                    sbuf[k][r, ds] = sbuf[k][r, ds] + g0
                    sbuf[k][r, ds1] = sbuf[k][r, ds1] + g1

            out_store(l, k).start()
        return carry

    lax.fori_loop(0, L // NBUF, outer, 0)
    out_store(L - 1, 1).wait()


@jax.jit
def _run(t4, idx4, tabp):
    mesh = plsc.VectorSubcoreMesh(core_axis_name="c", subcore_axis_name="s")
    scratch = [
        pltpu.VMEM((VAR_LEN,), jnp.int32),
        [pltpu.VMEM((BT, 128), jnp.int32) for _ in range(NBUF)],
        [pltpu.VMEM((BT, 256), jnp.float32) for _ in range(NBUF)],
        [pltpu.SemaphoreType.DMA for _ in range(NBUF)],
        [pltpu.SemaphoreType.DMA for _ in range(NBUF)],
        [pltpu.SemaphoreType.DMA for _ in range(NBUF)],
        pltpu.SemaphoreType.DMA,
    ]
    f = functools.partial(
        pl.kernel,
        out_type=jax.ShapeDtypeStruct((L * ET, BT, 1024), jnp.float32),
        mesh=mesh,
        scratch_types=scratch,
        compiler_params=pltpu.CompilerParams(
            use_tc_tiling_on_sc=False, needs_layout_passes=False
        ),
    )(_sc_body)
    return f(t4, idx4, tabp)


def kernel(src, variable_seq, emb_weight):
    t4 = (
        src.transpose(1, 2, 0)
        .reshape(L, ET, 8, BT, 128)
        .transpose(0, 1, 3, 2, 4)
        .reshape(L * ET, BT, 1024)
    )
    idx4 = (
        variable_seq.astype(jnp.int32)
        .transpose(1, 0)
        .reshape(LT, 8, BT, 128)
        .transpose(0, 2, 1, 3)
    )
    tabp = jax.lax.bitcast_convert_type(
        emb_weight.astype(jnp.bfloat16).reshape(VAR_LEN, NW, 2), jnp.int32
    ).transpose(1, 0)
    out4 = _run(t4, idx4, tabp)
    return (
        out4.reshape(L, ET, BT, 8, 128)
        .transpose(0, 1, 3, 2, 4)
        .reshape(L, EMBED, B)
        .transpose(2, 0, 1)
    )


# speedup vs baseline: 1.1535x; 1.1511x over previous
import functools

import jax
import jax.numpy as jnp
from jax import lax
from jax.experimental import pallas as pl
from jax.experimental.pallas import tpu as pltpu
from jax.experimental.pallas import tpu_sc as plsc

VAR_LEN = 100000
EMBED = 64
B = 4096
L = 200

_info = plsc.get_sparse_core_info()
NC, NS, NL = _info.num_cores, _info.num_subcores, _info.num_lanes
NW = NC * NS
EPW = EMBED // NW
BT = B // 128
ET = EMBED // 8
LT = L // 8
NBUF = 2
STLT = 5
NSTAGE = LT // STLT
SL = STLT * 8


def _sc_body(t_hbm, idx_hbm, tab_hbm, out_hbm, trow, sh_idx, idxb, sbuf,
             sem_i, sem_s, sem_o, sem_t):
    cid = lax.axis_index("c")
    sid = lax.axis_index("s")
    wid = sid * NC + cid

    for p in range(EPW):
        e = wid * EPW + p
        e_t = lax.div(e, 8)
        e_8 = lax.rem(e, 8)

        def idx_load(lr, k):
            return pltpu.make_async_copy(
                sh_idx.at[lax.div(lr, 8), :, lax.rem(lr, 8), :],
                idxb[k], sem_i[k],
            )

        def src_load(l, k):
            return pltpu.make_async_copy(
                t_hbm.at[l * ET + e_t, :, e_8, :], sbuf[k], sem_s[k]
            )

        def out_store(l, k):
            return pltpu.make_async_copy(
                sbuf[k], out_hbm.at[l * ET + e_t, :, e_8, :], sem_o[k]
            )

        pltpu.async_copy(tab_hbm.at[e], trow, sem_t).wait()

        def stage(s, carry):
            base = s * SL
            plsc.subcore_barrier()

            @pl.when(sid == 0)
            def _():
                pltpu.sync_copy(idx_hbm.at[pl.ds(s * STLT, STLT)], sh_idx)

            plsc.subcore_barrier()

            idx_load(0, 0).start()
            src_load(base, 0).start()

            def outer(h, carry2):
                for k in range(NBUF):
                    lr = NBUF * h + k
                    l = base + lr
                    kn = k ^ 1
                    idx_load(lr, k).wait()
                    src_load(l, k).wait()

                    @pl.when(lr > 0)
                    def _():
                        out_store(l - 1, kn).wait()

                    @pl.when(lr + 1 < SL)
                    def _():
                        idx_load(lr + 1, kn).start()
                        src_load(l + 1, kn).start()

                    @plsc.parallel_loop(0, BT, unroll=4)
                    def _(r):
                        for u in range(128 // NL):
                            iv = idxb[k][r, pl.ds(u * NL, NL)]
                            g = plsc.load_gather(trow, [iv])
                            sbuf[k][r, pl.ds(u * NL, NL)] = (
                                sbuf[k][r, pl.ds(u * NL, NL)] + g
                            )
                    out_store(l, k).start()
                return carry2

            lax.fori_loop(0, SL // NBUF, outer, 0)
            out_store(base + SL - 1, 1).wait()
            return carry

        lax.fori_loop(0, NSTAGE, stage, 0)


@jax.jit
def _run(t4, idx4, tab_t):
    mesh = plsc.VectorSubcoreMesh(core_axis_name="c", subcore_axis_name="s")
    scratch = [
        pltpu.VMEM((VAR_LEN,), jnp.float32),
        pltpu.VMEM_SHARED((STLT, BT, 8, 128), jnp.int32),
        [pltpu.VMEM((BT, 128), jnp.int32) for _ in range(NBUF)],
        [pltpu.VMEM((BT, 128), jnp.float32) for _ in range(NBUF)],
        [pltpu.SemaphoreType.DMA for _ in range(NBUF)],
        [pltpu.SemaphoreType.DMA for _ in range(NBUF)],
        [pltpu.SemaphoreType.DMA for _ in range(NBUF)],
        pltpu.SemaphoreType.DMA,
    ]
    f = functools.partial(
        pl.kernel,
        out_type=jax.ShapeDtypeStruct((L * ET, BT, 8, 128), jnp.float32),
        mesh=mesh,
        scratch_types=scratch,
        compiler_params=pltpu.CompilerParams(
            use_tc_tiling_on_sc=False, needs_layout_passes=False
        ),
    )(_sc_body)
    return f(t4, idx4, tab_t)


def kernel(src, variable_seq, emb_weight):
    t4 = (
        src.transpose(1, 2, 0)
        .reshape(L, ET, 8, BT, 128)
        .transpose(0, 1, 3, 2, 4)
        .reshape(L * ET, BT, 8, 128)
    )
    idx4 = (
        variable_seq.astype(jnp.int32)
        .transpose(1, 0)
        .reshape(LT, 8, BT, 128)
        .transpose(0, 2, 1, 3)
    )
    tab_t = emb_weight.transpose(1, 0)
    out4 = _run(t4, idx4, tab_t)
    return (
        out4.reshape(L, ET, BT, 8, 128)
        .transpose(0, 1, 3, 2, 4)
        .reshape(L, EMBED, B)
        .transpose(2, 0, 1)
    )


# speedup vs baseline: 1.1632x; 1.0084x over previous
import functools

import jax
import jax.numpy as jnp
from jax import lax
from jax.experimental import pallas as pl
from jax.experimental.pallas import tpu as pltpu
from jax.experimental.pallas import tpu_sc as plsc

VAR_LEN = 100000
EMBED = 64
B = 4096
L = 200

_info = plsc.get_sparse_core_info()
NC, NS, NL = _info.num_cores, _info.num_subcores, _info.num_lanes
NW = NC * NS
EPW = EMBED // NW
BT = B // 128
ET = EMBED // 8
LT = L // 8
NBUF = 2
STLT = 5
NSTAGE = LT // STLT
SL = STLT * 8


def _sc_body(t_hbm, idx_hbm, tab_hbm, out_hbm, trow, sh_idx, idxb, sbuf,
             sem_i, sem_s, sem_o, sem_t):
    cid = lax.axis_index("c")
    sid = lax.axis_index("s")
    wid = sid * NC + cid

    for p in range(EPW):
        e = wid * EPW + p
        e_t = lax.div(e, 8)
        e_8 = lax.rem(e, 8)

        def idx_load(lr, k):
            return pltpu.make_async_copy(
                sh_idx.at[lax.div(lr, 8), :, lax.rem(lr, 8), :],
                idxb[k], sem_i[k],
            )

        def src_load(l, k):
            return pltpu.make_async_copy(
                t_hbm.at[l * ET + e_t, :, e_8, :], sbuf[k], sem_s[k]
            )

        def out_store(l, k):
            return pltpu.make_async_copy(
                sbuf[k], out_hbm.at[l * ET + e_t, :, e_8, :], sem_o[k]
            )

        trow_copy = pltpu.make_async_copy(tab_hbm.at[e], trow, sem_t)
        trow_copy.start()

        def stage(s, carry):
            base = s * SL
            plsc.subcore_barrier()

            @pl.when(sid == 0)
            def _():
                pltpu.sync_copy(idx_hbm.at[pl.ds(s * STLT, STLT)], sh_idx)

            plsc.subcore_barrier()

            idx_load(0, 0).start()
            src_load(base, 0).start()

            @pl.when(s == 0)
            def _():
                trow_copy.wait()

            def outer(h, carry2):
                for k in range(NBUF):
                    lr = NBUF * h + k
                    l = base + lr
                    kn = k ^ 1
                    idx_load(lr, k).wait()
                    src_load(l, k).wait()

                    @pl.when(lr > 0)
                    def _():
                        out_store(l - 1, kn).wait()

                    @pl.when(lr + 1 < SL)
                    def _():
                        idx_load(lr + 1, kn).start()
                        src_load(l + 1, kn).start()

                    @plsc.parallel_loop(0, BT, unroll=4)
                    def _(r):
                        for u in range(128 // NL):
                            iv = idxb[k][r, pl.ds(u * NL, NL)]
                            g = plsc.load_gather(trow, [iv])
                            sbuf[k][r, pl.ds(u * NL, NL)] = (
                                sbuf[k][r, pl.ds(u * NL, NL)] + g
                            )
                    out_store(l, k).start()
                return carry2

            lax.fori_loop(0, SL // NBUF, outer, 0)
            out_store(base + SL - 1, 1).wait()
            return carry

        lax.fori_loop(0, NSTAGE, stage, 0)


@jax.jit
def _run(t4, idx4, tab_t):
    mesh = plsc.VectorSubcoreMesh(core_axis_name="c", subcore_axis_name="s")
    scratch = [
        pltpu.VMEM((VAR_LEN,), jnp.float32),
        pltpu.VMEM_SHARED((STLT, BT, 8, 128), jnp.int32),
        [pltpu.VMEM((BT, 128), jnp.int32) for _ in range(NBUF)],
        [pltpu.VMEM((BT, 128), jnp.float32) for _ in range(NBUF)],
        [pltpu.SemaphoreType.DMA for _ in range(NBUF)],
        [pltpu.SemaphoreType.DMA for _ in range(NBUF)],
        [pltpu.SemaphoreType.DMA for _ in range(NBUF)],
        pltpu.SemaphoreType.DMA,
    ]
    f = functools.partial(
        pl.kernel,
        out_type=jax.ShapeDtypeStruct((L * ET, BT, 8, 128), jnp.float32),
        mesh=mesh,
        scratch_types=scratch,
        compiler_params=pltpu.CompilerParams(
            use_tc_tiling_on_sc=False, needs_layout_passes=False
        ),
    )(_sc_body)
    return f(t4, idx4, tab_t)


def kernel(src, variable_seq, emb_weight):
    t4 = (
        src.transpose(1, 2, 0)
        .reshape(L, ET, 8, BT, 128)
        .transpose(0, 1, 3, 2, 4)
        .reshape(L * ET, BT, 8, 128)
    )
    idx4 = (
        variable_seq.astype(jnp.int32)
        .transpose(1, 0)
        .reshape(LT, 8, BT, 128)
        .transpose(0, 2, 1, 3)
    )
    tab_t = emb_weight.transpose(1, 0)
    out4 = _run(t4, idx4, tab_t)
    return (
        out4.reshape(L, ET, BT, 8, 128)
        .transpose(0, 1, 3, 2, 4)
        .reshape(L, EMBED, B)
        .transpose(2, 0, 1)
    )
